# baseline (speedup 1.0000x reference)
import functools

import jax
import jax.numpy as jnp
from jax import lax
from jax.experimental import pallas as pl
from jax.experimental.pallas import tpu as pltpu
from jax.experimental.pallas import tpu_sc as plsc

LANES = 16
NBUF = 4
DEPTH = 3


def _build_sc_kernel(B, L, V, D):
    info = plsc.get_sparse_core_info()
    NC, NS = info.num_cores, info.num_subcores
    NW = NC * NS
    assert B % NW == 0
    rows_per_w = B // NW
    toks_per_w = rows_per_w * L
    assert rows_per_w % NBUF == 0
    n_steps = rows_per_w // NBUF
    assert D % LANES == 0
    KD = D // LANES
    NG = (L + LANES - 1) // LANES
    g_starts = [g * LANES for g in range(NG - 1)] + [L - LANES]
    LPAD = L + 8

    mesh = plsc.VectorSubcoreMesh(core_axis_name="c", subcore_axis_name="s")

    @functools.partial(
        pl.kernel,
        mesh=mesh,
        compiler_params=pltpu.CompilerParams(use_tc_tiling_on_sc=False),
        out_type=jax.ShapeDtypeStruct((B, L, D), jnp.float32),
        scratch_types=[
            pltpu.VMEM((rows_per_w + LANES,), jnp.int32),
            pltpu.VMEM((L, D), jnp.float32),
            pltpu.VMEM((toks_per_w,), jnp.int32),
            [pltpu.VMEM((LPAD, D), jnp.float32) for _ in range(NBUF)],
            [pltpu.SemaphoreType.DMA for _ in range(NBUF)],
            [pltpu.SemaphoreType.DMA for _ in range(NBUF)],
        ],
    )
    def k(x_hbm, lens_hbm, emb_hbm, pe_hbm, out_hbm,
          lens_v, pem_v, idx_all, rows, sem_g, sem_s):
        wid = lax.axis_index("s") * NC + lax.axis_index("c")
        base_row = wid * rows_per_w
        base_tok = wid * toks_per_w

        pltpu.sync_copy(lens_hbm.at[pl.ds(base_row, rows_per_w)],
                        lens_v.at[pl.ds(0, rows_per_w)])
        pltpu.sync_copy(pe_hbm, pem_v)
        pltpu.sync_copy(x_hbm.at[pl.ds(base_tok, toks_per_w)], idx_all)

        zero = jnp.int32(0)
        zeros16 = jnp.zeros((LANES,), jnp.int32)

        def pe_scale(i, carry):
            for kk in range(KD):
                s = pl.ds(kk * LANES, LANES)
                pem_v[i, s] = pem_v[i, s] * jnp.float32(0.125)
            return carry

        lax.fori_loop(0, L, pe_scale, zero)

        def gather_start(r, b):
            for gs in g_starts:
                idx16 = idx_all[pl.ds(r * L + gs, LANES)]
                pltpu.async_copy(emb_hbm.at[idx16],
                                 rows[b].at[pl.ds(gs, LANES)],
                                 sem_g[b])

        def gather_wait(r, b):
            pltpu.make_async_copy(emb_hbm.at[pl.ds(0, NG * LANES)],
                                  rows[b].at[pl.ds(0, NG * LANES)],
                                  sem_g[b]).wait()

        def store_start(r, b):
            pltpu.async_copy(rows[b].at[pl.ds(0, L)],
                             out_hbm.at[base_row + r], sem_s[b])

        def store_wait(r, b):
            pltpu.make_async_copy(rows[b].at[pl.ds(0, L)],
                                  out_hbm.at[base_row + r], sem_s[b]).wait()

        def compute(r, b):
            rb = r & jnp.int32(~15)
            lens16 = lens_v[pl.ds(rb, LANES)]
            ln = lens16.at[zeros16 + (r - rb)].get(mode="promise_in_bounds")

            @plsc.parallel_loop(0, L, unroll=2)
            def token(t):
                mf = jnp.where(zeros16 + t < ln, jnp.float32(8.0),
                               jnp.float32(0.0))
                for kk in range(KD):
                    s = pl.ds(kk * LANES, LANES)
                    rows[b][t, s] = (rows[b][t, s] + pem_v[t, s]) * mf

        for b in range(DEPTH):
            gather_start(jnp.int32(b), b)

        def step(s, carry):
            for cc in range(NBUF):
                b = cc
                r = NBUF * s + cc
                gather_wait(r, b)
                b2 = (cc + DEPTH) % NBUF
                @pl.when(r + DEPTH < rows_per_w)
                def _():
                    @pl.when(r >= NBUF - DEPTH)
                    def _():
                        store_wait(r + DEPTH - NBUF, b2)
                    gather_start(r + DEPTH, b2)
                compute(r, b)
                store_start(r, b)
            return carry

        lax.fori_loop(0, n_steps, step, zero)

        for i in range(NBUF):
            store_wait(rows_per_w - NBUF + i, i)

    return k


def kernel(x, input_lengths, embedding_weight, pos_enc):
    B, L = x.shape
    V, D = embedding_weight.shape
    k = _build_sc_kernel(B, L, V, D)
    return k(x.reshape(-1), input_lengths, embedding_weight, pos_enc)


# speedup vs baseline: 1.0385x; 1.0385x over previous
import functools

import jax
import jax.numpy as jnp
from jax import lax
from jax.experimental import pallas as pl
from jax.experimental.pallas import tpu as pltpu
from jax.experimental.pallas import tpu_sc as plsc

LANES = 16
NBUF = 4
DEPTH = 2


def _build_sc_kernel(B, L, V, D):
    info = plsc.get_sparse_core_info()
    NC, NS = info.num_cores, info.num_subcores
    NW = NC * NS
    assert B % NW == 0
    rows_per_w = B // NW
    toks_per_w = rows_per_w * L
    assert rows_per_w % NBUF == 0
    n_steps = rows_per_w // NBUF
    assert D % LANES == 0
    KD = D // LANES
    NG = (L + LANES - 1) // LANES
    g_starts = [g * LANES for g in range(NG - 1)] + [L - LANES]
    LPAD = L + 8

    mesh = plsc.VectorSubcoreMesh(core_axis_name="c", subcore_axis_name="s")

    @functools.partial(
        pl.kernel,
        mesh=mesh,
        compiler_params=pltpu.CompilerParams(use_tc_tiling_on_sc=False),
        out_type=jax.ShapeDtypeStruct((B, L, D), jnp.float32),
        scratch_types=[
            pltpu.VMEM((rows_per_w + LANES,), jnp.int32),
            pltpu.VMEM((L, D), jnp.float32),
            pltpu.VMEM((toks_per_w,), jnp.int32),
            [pltpu.VMEM((LPAD, D), jnp.float32) for _ in range(NBUF)],
            [pltpu.SemaphoreType.DMA for _ in range(NBUF)],
            [pltpu.SemaphoreType.DMA for _ in range(NBUF)],
        ],
    )
    def k(x_hbm, lens_hbm, emb_hbm, pe_hbm, out_hbm,
          lens_v, pem_v, idx_all, rows, sem_g, sem_s):
        wid = lax.axis_index("s") * NC + lax.axis_index("c")
        base_row = wid * rows_per_w
        base_tok = wid * toks_per_w

        pltpu.sync_copy(lens_hbm.at[pl.ds(base_row, rows_per_w)],
                        lens_v.at[pl.ds(0, rows_per_w)])
        pltpu.sync_copy(pe_hbm, pem_v)
        pltpu.sync_copy(x_hbm.at[pl.ds(base_tok, toks_per_w)], idx_all)

        zero = jnp.int32(0)
        zeros16 = jnp.zeros((LANES,), jnp.int32)

        def pe_scale(i, carry):
            for kk in range(KD):
                s = pl.ds(kk * LANES, LANES)
                pem_v[i, s] = pem_v[i, s] * jnp.float32(0.125)
            return carry

        lax.fori_loop(0, L, pe_scale, zero)

        def gather_start(r, b):
            for gs in g_starts:
                idx16 = idx_all[pl.ds(r * L + gs, LANES)]
                pltpu.async_copy(emb_hbm.at[idx16],
                                 rows[b].at[pl.ds(gs, LANES)],
                                 sem_g[b])

        def gather_wait(r, b):
            pltpu.make_async_copy(emb_hbm.at[pl.ds(0, NG * LANES)],
                                  rows[b].at[pl.ds(0, NG * LANES)],
                                  sem_g[b]).wait()

        def store_start(r, b):
            pltpu.async_copy(rows[b].at[pl.ds(0, L)],
                             out_hbm.at[base_row + r], sem_s[b])

        def store_wait(r, b):
            pltpu.make_async_copy(rows[b].at[pl.ds(0, L)],
                                  out_hbm.at[base_row + r], sem_s[b]).wait()

        def compute(r, b):
            rb = r & jnp.int32(~15)
            lens16 = lens_v[pl.ds(rb, LANES)]
            ln = lens16.at[zeros16 + (r - rb)].get(mode="promise_in_bounds")

            @plsc.parallel_loop(0, L, unroll=2)
            def token(t):
                mf = jnp.where(zeros16 + t < ln, jnp.float32(8.0),
                               jnp.float32(0.0))
                for kk in range(KD):
                    s = pl.ds(kk * LANES, LANES)
                    rows[b][t, s] = (rows[b][t, s] + pem_v[t, s]) * mf

        for b in range(DEPTH):
            gather_start(jnp.int32(b), b)

        def step(s, carry):
            for cc in range(NBUF):
                b = cc
                r = NBUF * s + cc
                gather_wait(r, b)
                b2 = (cc + DEPTH) % NBUF
                @pl.when(r + DEPTH < rows_per_w)
                def _():
                    @pl.when(r >= NBUF - DEPTH)
                    def _():
                        store_wait(r + DEPTH - NBUF, b2)
                    gather_start(r + DEPTH, b2)
                compute(r, b)
                store_start(r, b)
            return carry

        lax.fori_loop(0, n_steps, step, zero)

        for i in range(NBUF):
            store_wait(rows_per_w - NBUF + i, i)

    return k


def kernel(x, input_lengths, embedding_weight, pos_enc):
    B, L = x.shape
    V, D = embedding_weight.shape
    k = _build_sc_kernel(B, L, V, D)
    return k(x.reshape(-1), input_lengths, embedding_weight, pos_enc)
